# baseline (speedup 1.0000x reference)
import functools

import jax
import jax.numpy as jnp
from jax import lax
from jax.experimental import pallas as pl
from jax.experimental.pallas import tpu as pltpu
from jax.experimental.pallas import tpu_sc as plsc

D_MODEL = 64
SCALE = 8.0

_NC = 2
_NS = 16
_NW = _NC * _NS
_LANES = 16

_N = 16384 * 50
_B_PER_W = _N // _NW
_CHUNK = 512
_NCHUNK = _B_PER_W // _CHUNK


_mesh = plsc.VectorSubcoreMesh(core_axis_name="c", subcore_axis_name="s")


@functools.partial(
    pl.kernel,
    out_type=jax.ShapeDtypeStruct((_N, D_MODEL), jnp.float32),
    mesh=_mesh,
    scratch_types=[
        pltpu.VMEM((_CHUNK,), jnp.int32),
        pltpu.VMEM((_CHUNK, D_MODEL), jnp.float32),
        pltpu.SemaphoreType.DMA,
    ],
    compiler_params=pltpu.CompilerParams(use_tc_tiling_on_sc=False),
)
def _embed(idx_hbm, table_hbm, out_hbm, idx_v, rows_v, sem):
    wid = lax.axis_index("s") * _NC + lax.axis_index("c")
    base = wid * _B_PER_W

    def chunk_body(ci, carry):
        off = base + ci * _CHUNK
        pltpu.sync_copy(idx_hbm.at[pl.ds(off, _CHUNK)], idx_v)
        pltpu.async_copy(table_hbm.at[idx_v], rows_v, sem).wait()

        def row_body(i, c2):
            for j in range(D_MODEL // _LANES):
                s = pl.ds(j * _LANES, _LANES)
                rows_v[i, s] = rows_v[i, s] * SCALE
            return c2

        lax.fori_loop(0, _CHUNK, row_body, 0)
        pltpu.sync_copy(rows_v, out_hbm.at[pl.ds(off, _CHUNK)])
        return carry

    lax.fori_loop(0, _NCHUNK, chunk_body, 0)


def kernel(x, table):
    idx = x.reshape(-1)
    out = _embed(idx, table)
    return out.reshape(x.shape[0], x.shape[1], D_MODEL)


# speedup vs baseline: 1.1185x; 1.1185x over previous
import functools

import jax
import jax.numpy as jnp
from jax import lax
from jax.experimental import pallas as pl
from jax.experimental.pallas import tpu as pltpu
from jax.experimental.pallas import tpu_sc as plsc

D_MODEL = 64
SCALE = 8.0

_NC = 2
_NS = 16
_NW = _NC * _NS
_LANES = 16

_N = 16384 * 50
_B_PER_W = _N // _NW
_CHUNK = 512
_NCHUNK = _B_PER_W // _CHUNK


_mesh = plsc.VectorSubcoreMesh(core_axis_name="c", subcore_axis_name="s")


@functools.partial(
    pl.kernel,
    out_type=jax.ShapeDtypeStruct((_N, D_MODEL), jnp.float32),
    mesh=_mesh,
    scratch_types=[
        pltpu.VMEM((_CHUNK,), jnp.int32),
        pltpu.VMEM((_CHUNK,), jnp.int32),
        pltpu.VMEM((_CHUNK, D_MODEL), jnp.float32),
        pltpu.VMEM((_CHUNK, D_MODEL), jnp.float32),
        pltpu.SemaphoreType.DMA,
        pltpu.SemaphoreType.DMA,
        pltpu.SemaphoreType.DMA,
        pltpu.SemaphoreType.DMA,
    ],
    compiler_params=pltpu.CompilerParams(use_tc_tiling_on_sc=False),
)
def _embed(idx_hbm, table_hbm, out_hbm, idx_v0, idx_v1, rows_v0, rows_v1,
           g0, g1, o0, o1):
    wid = lax.axis_index("s") * _NC + lax.axis_index("c")
    base = wid * _B_PER_W
    idx_v = (idx_v0, idx_v1)
    rows_v = (rows_v0, rows_v1)
    gsem = (g0, g1)
    osem = (o0, o1)

    def idx_copy(ci, b):
        pltpu.sync_copy(idx_hbm.at[pl.ds(base + ci * _CHUNK, _CHUNK)],
                        idx_v[b])

    def gather_start(b):
        pltpu.async_copy(table_hbm.at[idx_v[b]], rows_v[b], gsem[b])

    def gather_wait(b):
        pltpu.make_async_copy(table_hbm.at[idx_v[b]], rows_v[b],
                              gsem[b]).wait()

    def out_start(ci, b):
        pltpu.async_copy(rows_v[b],
                         out_hbm.at[pl.ds(base + ci * _CHUNK, _CHUNK)],
                         osem[b])

    def out_wait(b):
        pltpu.make_async_copy(rows_v[b],
                              out_hbm.at[pl.ds(base, _CHUNK)],
                              osem[b]).wait()

    def scale(b):
        ref = rows_v[b]

        @plsc.parallel_loop(0, _CHUNK, unroll=4)
        def _scale(i):
            for j in range(D_MODEL // _LANES):
                s = pl.ds(j * _LANES, _LANES)
                ref[i, s] = ref[i, s] * SCALE

    idx_copy(0, 0)
    gather_start(0)

    idx_copy(1, 1)
    gather_start(1)
    gather_wait(0)
    scale(0)
    out_start(0, 0)

    def pair_body(k, carry):
        for (g_off, b) in ((1, 1), (2, 0)):
            g = 2 * k + g_off
            nb = 1 - b
            out_wait(nb)
            idx_copy(g + 1, nb)
            gather_start(nb)
            gather_wait(b)
            scale(b)
            out_start(g, b)
        return carry

    lax.fori_loop(0, (_NCHUNK - 2) // 2, pair_body, 0)

    gather_wait(1)
    scale(1)
    out_start(_NCHUNK - 1, 1)
    out_wait(0)
    out_wait(1)


def kernel(x, table):
    idx = x.reshape(-1)
    out = _embed(idx, table)
    return out.reshape(x.shape[0], x.shape[1], D_MODEL)
